# Initial kernel scaffold; baseline (speedup 1.0000x reference)
import jax
import jax.numpy as jnp
from jax.experimental import pallas as pl


def kernel(rbf, sph, idx_sph, weight):
    raise NotImplementedError("write your pallas kernel here")


# speedup vs baseline: 29.3233x; 29.3233x over previous
import functools

import jax
import jax.numpy as jnp
from jax import lax
from jax.experimental import pallas as pl
from jax.experimental.pallas import tpu as pltpu
from jax.experimental.pallas import tpu_sc as plsc

E = 100000
T = 300000
R = 128
S = 8
M = 32

NC = 2
NS = 16
NW = NC * NS
C = 128

CHUNKS_PER_W = (T + NW * C - 1) // (NW * C)
T_PAD = NW * C * CHUNKS_PER_W


def _mm_body(a_ref, b_ref, o_ref):
    o_ref[...] = jnp.dot(a_ref[...], b_ref[...],
                         preferred_element_type=jnp.float32)


def _matmul(rbf, wp):
    blk = 2000
    grid = (E // blk,)
    return pl.pallas_call(
        _mm_body,
        grid=grid,
        in_specs=[
            pl.BlockSpec((blk, R), lambda i: (i, 0)),
            pl.BlockSpec((R, S * M), lambda i: (0, 0)),
        ],
        out_specs=pl.BlockSpec((blk, S * M), lambda i: (i, 0)),
        out_shape=jax.ShapeDtypeStruct((E, S * M), jnp.float32),
    )(rbf, wp)


def _sc_body(u_hbm, idx_hbm, sph_hbm, out_hbm,
             idx_v, sph_v, rows_v, out_v, sem):
    wid = lax.axis_index("s") * NC + lax.axis_index("c")
    base = wid * (CHUNKS_PER_W * C)

    def chunk_step(g, _):
        off = base + g * C
        pltpu.sync_copy(idx_hbm.at[pl.ds(off, C)], idx_v)
        pltpu.sync_copy(sph_hbm.at[pl.ds(off * S, C * S)], sph_v)
        pltpu.async_copy(u_hbm.at[idx_v], rows_v, sem).wait()

        def tri(i, _):
            sv = sph_v[pl.ds(i * 16, 16)]
            for t2 in range(2):
                j = i * 2 + t2
                acc0 = jnp.zeros((16,), jnp.float32)
                acc1 = jnp.zeros((16,), jnp.float32)
                for b in range(S):
                    s = lax.broadcast(sv[t2 * S + b], (16,))
                    acc0 = acc0 + s * rows_v[j, pl.ds(b * M, 16)]
                    acc1 = acc1 + s * rows_v[j, pl.ds(b * M + 16, 16)]
                out_v[j, pl.ds(0, 16)] = acc0
                out_v[j, pl.ds(16, 16)] = acc1
            return 0

        lax.fori_loop(0, C // 2, tri, 0)
        pltpu.sync_copy(out_v, out_hbm.at[pl.ds(off, C)])
        return 0

    lax.fori_loop(0, CHUNKS_PER_W, chunk_step, 0)


_sc_lookup = functools.partial(
    pl.kernel,
    out_type=jax.ShapeDtypeStruct((T_PAD, M), jnp.float32),
    mesh=plsc.VectorSubcoreMesh(core_axis_name="c", subcore_axis_name="s"),
    scratch_types=[
        pltpu.VMEM((C,), jnp.int32),
        pltpu.VMEM((C * S,), jnp.float32),
        pltpu.VMEM((C, S * M), jnp.float32),
        pltpu.VMEM((C, M), jnp.float32),
        pltpu.SemaphoreType.DMA,
    ],
)(_sc_body)


def kernel(rbf, sph, idx_sph, weight):
    w2 = weight.reshape(R, S * M)
    wp = w2.reshape(R, M, S).transpose(0, 2, 1).reshape(R, S * M)
    u = _matmul(rbf, wp)

    idx_p = jnp.zeros((T_PAD,), jnp.int32).at[:T].set(idx_sph)
    sph_p = jnp.zeros((T_PAD * S,), jnp.float32).at[:T * S].set(sph.reshape(-1))

    out = _sc_lookup(u, idx_p, sph_p)
    return out[:T]
